# baseline (speedup 1.0000x reference)
import functools

import jax
import jax.numpy as jnp
from jax import lax
from jax.experimental import pallas as pl
from jax.experimental.pallas import tpu as pltpu
from jax.experimental.pallas import tpu_sc as plsc

NC = 2
NS = 16
NW = NC * NS
L = 16
NACC = 5

B, H, W, C = 8, 512, 512, 3
N_PIX = B * H * W
PIX_W = N_PIX // NW
HROWS_W = PIX_W // W
N_CHUNK = 8
CH_H = HROWS_W // N_CHUNK
CHUNK_PIX = CH_H * W
GROUPS = CHUNK_PIX // L
IROWS = CHUNK_PIX // 128

_params = pltpu.CompilerParams(use_tc_tiling_on_sc=True,
                               needs_layout_passes=False)


def _make_kernels():
    mesh = plsc.VectorSubcoreMesh(core_axis_name="c", subcore_axis_name="s",
                                  num_cores=NC, num_subcores=NS)

    @functools.partial(
        pl.kernel,
        out_type=jax.ShapeDtypeStruct((NW, 8, 128), jnp.float32),
        mesh=mesh,
        compiler_params=_params,
        scratch_types=[
            pltpu.VMEM((3 * CH_H, W), jnp.float32),
            pltpu.VMEM((3 * CH_H, W), jnp.float32),
            pltpu.VMEM((IROWS, 128), jnp.int32),
            pltpu.VMEM((IROWS, 128), jnp.int32),
            pltpu.VMEM((8, 128), jnp.float32),
            pltpu.SemaphoreType.DMA,
            pltpu.SemaphoreType.DMA,
            pltpu.SemaphoreType.DMA,
            pltpu.SemaphoreType.DMA,
        ],
    )
    def phase1(feats_hbm, inst_hbm, part_hbm, fbuf0, fbuf1, ibuf0, ibuf1,
               obuf, semf0, semf1, semi0, semi1):
        wid = lax.axis_index("s") * NC + lax.axis_index("c")
        b = wid // 4
        q = wid % 4
        h0 = q * HROWS_W
        fbase = b * (C * H) + h0
        irow0 = wid * (PIX_W // 128)
        fbuf = [fbuf0, fbuf1]
        ibuf = [ibuf0, ibuf1]
        semf = [semf0, semf1]
        semi = [semi0, semi1]

        def start(g, slot):
            hs = []
            for c in range(C):
                r = fbase + c * H + g * CH_H
                hs.append(pltpu.async_copy(
                    feats_hbm.at[pl.ds(r, CH_H), :],
                    fbuf[slot].at[pl.ds(c * CH_H, CH_H), :], semf[slot]))
            hs.append(pltpu.async_copy(
                inst_hbm.at[pl.ds(irow0 + g * IROWS, IROWS), :],
                ibuf[slot], semi[slot]))
            return hs

        pend = [None, None]
        pend[0] = start(0, 0)
        zeros = jnp.zeros((L,), jnp.float32)
        carry = (zeros, zeros, zeros, zeros, zeros)
        for g in range(N_CHUNK):
            slot = g % 2
            if g + 1 < N_CHUNK:
                pend[(g + 1) % 2] = start(g + 1, (g + 1) % 2)
            for hdl in pend[slot]:
                hdl.wait()
            fslot = fbuf[slot]
            islot = ibuf[slot]

            def body(i, acc, fslot=fslot, islot=islot):
                t0, t1, t2, c1, c2 = acc
                hr = i >> 5
                wc = pl.multiple_of((i & 31) << 4, 16)
                ir = i >> 3
                ic = pl.multiple_of((i & 7) << 4, 16)
                cv = islot[ir, pl.ds(ic, L)].astype(jnp.float32)
                t = (fslot[hr, pl.ds(wc, L)]
                     + fslot[hr + CH_H, pl.ds(wc, L)]
                     + fslot[hr + 2 * CH_H, pl.ds(wc, L)])
                x = t * cv
                return (t0 + t, t1 + x, t2 + x * cv, c1 + cv, c2 + cv * cv)

            carry = lax.fori_loop(0, GROUPS, body, carry, unroll=4)

        for a in range(NACC):
            obuf[0, pl.ds(a * L, L)] = carry[a]
        pltpu.sync_copy(obuf, part_hbm.at[wid])

    def phase2_body(iref, pref, oref):
        part = pref[...]
        s = jnp.sum(part[:, 0, :], axis=0)
        t0 = jnp.sum(s[0 * L:1 * L])
        t1 = jnp.sum(s[1 * L:2 * L])
        t2 = jnp.sum(s[2 * L:3 * L])
        c1 = jnp.sum(s[3 * L:4 * L])
        c2 = jnp.sum(s[4 * L:5 * L])
        n_elems = jnp.float32(N_PIX * 3)
        s2 = (t2 - t1) * 0.5
        s1 = t1 - 2.0 * s2
        s0 = t0 - s1 - s2
        n2 = 3.0 * (c2 - c1) * 0.5
        n1 = 3.0 * c1 - 2.0 * n2
        n0 = n_elems - n1 - n2
        m0 = s0 / n0
        m1 = s1 / n1
        m2 = s2 / n2
        cls = iref[0]
        oref[...] = jnp.where(cls == 0, m0,
                              jnp.where(cls == 1, m1, m2)).astype(jnp.float32)

    phase2 = pl.pallas_call(
        phase2_body,
        grid=(B * C,),
        in_specs=[
            pl.BlockSpec((1, H, W), lambda i: (i // C, 0, 0)),
            pl.BlockSpec((NW, 8, 128), lambda i: (0, 0, 0)),
        ],
        out_specs=pl.BlockSpec((H, W), lambda i: (i, 0)),
        out_shape=jax.ShapeDtypeStruct((B * C * H, W), jnp.float32),
        compiler_params=pltpu.CompilerParams(
            dimension_semantics=("arbitrary",)),
    )

    def run(feats, inst):
        ft = feats.transpose(0, 3, 1, 2).reshape(B * C * H, W)
        ii = inst.reshape(N_PIX // 128, 128)
        inst3d = inst.reshape(B, H, W)
        part = phase1(ft, ii)
        out2d = phase2(inst3d, part)
        return out2d.reshape(B, C, H, W).transpose(0, 2, 3, 1)

    return run


_make_kernels_cached = functools.lru_cache(maxsize=None)(_make_kernels)


@jax.jit
def kernel(feats, inst):
    return _make_kernels_cached()(feats, inst)


# speedup vs baseline: 1.1719x; 1.1719x over previous
import functools

import jax
import jax.numpy as jnp
from jax import lax
from jax.experimental import pallas as pl
from jax.experimental.pallas import tpu as pltpu
from jax.experimental.pallas import tpu_sc as plsc

NC = 2
NS = 16
NW = NC * NS
L = 16
NACC = 5

B, H, W, C = 8, 512, 512, 3
N_PIX = B * H * W
PIX_W = N_PIX // NW
HROWS_W = PIX_W // W
N_CHUNK = 8
CH_H = HROWS_W // N_CHUNK
CHUNK_PIX = CH_H * W
GROUPS = CHUNK_PIX // L
IROWS = CHUNK_PIX // 128

_params = pltpu.CompilerParams(use_tc_tiling_on_sc=True,
                               needs_layout_passes=False)


def _make_kernels():
    mesh = plsc.VectorSubcoreMesh(core_axis_name="c", subcore_axis_name="s",
                                  num_cores=NC, num_subcores=NS)

    @functools.partial(
        pl.kernel,
        out_type=jax.ShapeDtypeStruct((NW, 8, 128), jnp.float32),
        mesh=mesh,
        compiler_params=_params,
        scratch_types=[
            pltpu.VMEM((3 * CH_H, W), jnp.float32),
            pltpu.VMEM((3 * CH_H, W), jnp.float32),
            pltpu.VMEM((IROWS, 128), jnp.int32),
            pltpu.VMEM((IROWS, 128), jnp.int32),
            pltpu.VMEM((8, 128), jnp.float32),
            pltpu.SemaphoreType.DMA,
            pltpu.SemaphoreType.DMA,
            pltpu.SemaphoreType.DMA,
            pltpu.SemaphoreType.DMA,
        ],
    )
    def phase1(feats_hbm, inst_hbm, part_hbm, fbuf0, fbuf1, ibuf0, ibuf1,
               obuf, semf0, semf1, semi0, semi1):
        wid = lax.axis_index("s") * NC + lax.axis_index("c")
        b = wid // 4
        q = wid % 4
        h0 = q * HROWS_W
        fbase = b * (C * H) + h0
        irow0 = wid * (PIX_W // 128)
        fbuf = [fbuf0, fbuf1]
        ibuf = [ibuf0, ibuf1]
        semf = [semf0, semf1]
        semi = [semi0, semi1]

        def start(g, slot):
            hs = []
            for c in range(C):
                r = fbase + c * H + g * CH_H
                hs.append(pltpu.async_copy(
                    feats_hbm.at[pl.ds(r, CH_H), :],
                    fbuf[slot].at[pl.ds(c * CH_H, CH_H), :], semf[slot]))
            hs.append(pltpu.async_copy(
                inst_hbm.at[pl.ds(irow0 + g * IROWS, IROWS), :],
                ibuf[slot], semi[slot]))
            return hs

        pend = [None, None]
        pend[0] = start(0, 0)
        zeros = jnp.zeros((L,), jnp.float32)
        carry = (zeros, zeros, zeros, zeros, zeros)
        for g in range(N_CHUNK):
            slot = g % 2
            if g + 1 < N_CHUNK:
                pend[(g + 1) % 2] = start(g + 1, (g + 1) % 2)
            for hdl in pend[slot]:
                hdl.wait()
            fslot = fbuf[slot]
            islot = ibuf[slot]

            def body(i, acc, fslot=fslot, islot=islot):
                t0, t1, t2, c1, c2 = acc
                hr = i >> 5
                wc = pl.multiple_of((i & 31) << 4, 16)
                ir = i >> 3
                ic = pl.multiple_of((i & 7) << 4, 16)
                cv = islot[ir, pl.ds(ic, L)].astype(jnp.float32)
                t = (fslot[hr, pl.ds(wc, L)]
                     + fslot[hr + CH_H, pl.ds(wc, L)]
                     + fslot[hr + 2 * CH_H, pl.ds(wc, L)])
                x = t * cv
                return (t0 + t, t1 + x, t2 + x * cv, c1 + cv, c2 + cv * cv)

            carry = lax.fori_loop(0, GROUPS, body, carry, unroll=4)

        for a in range(NACC):
            obuf[0, pl.ds(a * L, L)] = carry[a]
        pltpu.sync_copy(obuf, part_hbm.at[wid])

    def phase2_body(iref, pref, oref):
        part = pref[...]
        s = jnp.sum(part[:, 0, :], axis=0)
        t0 = jnp.sum(s[0 * L:1 * L])
        t1 = jnp.sum(s[1 * L:2 * L])
        t2 = jnp.sum(s[2 * L:3 * L])
        c1 = jnp.sum(s[3 * L:4 * L])
        c2 = jnp.sum(s[4 * L:5 * L])
        n_elems = jnp.float32(N_PIX * 3)
        s2 = (t2 - t1) * 0.5
        s1 = t1 - 2.0 * s2
        s0 = t0 - s1 - s2
        n2 = 3.0 * (c2 - c1) * 0.5
        n1 = 3.0 * c1 - 2.0 * n2
        n0 = n_elems - n1 - n2
        m0 = s0 / n0
        m1 = s1 / n1
        m2 = s2 / n2
        cls = iref[0]
        v = jnp.where(cls == 0, m0,
                      jnp.where(cls == 1, m1, m2)).astype(jnp.float32)
        for c in range(C):
            oref[pl.ds(c * H, H), :] = v

    phase2 = pl.pallas_call(
        phase2_body,
        grid=(B,),
        in_specs=[
            pl.BlockSpec((1, H, W), lambda i: (i, 0, 0)),
            pl.BlockSpec((NW, 8, 128), lambda i: (0, 0, 0)),
        ],
        out_specs=pl.BlockSpec((C * H, W), lambda i: (i, 0)),
        out_shape=jax.ShapeDtypeStruct((B * C * H, W), jnp.float32),
        compiler_params=pltpu.CompilerParams(
            dimension_semantics=("parallel",)),
    )

    def run(feats, inst):
        ft = feats.transpose(0, 3, 1, 2).reshape(B * C * H, W)
        ii = inst.reshape(N_PIX // 128, 128)
        inst3d = inst.reshape(B, H, W)
        part = phase1(ft, ii)
        out2d = phase2(inst3d, part)
        return out2d.reshape(B, C, H, W).transpose(0, 2, 3, 1)

    return run


_make_kernels_cached = functools.lru_cache(maxsize=None)(_make_kernels)


@jax.jit
def kernel(feats, inst):
    return _make_kernels_cached()(feats, inst)


# speedup vs baseline: 1.3863x; 1.1830x over previous
import functools

import jax
import jax.numpy as jnp
from jax import lax
from jax.experimental import pallas as pl
from jax.experimental.pallas import tpu as pltpu
from jax.experimental.pallas import tpu_sc as plsc

NC = 2
NS = 16
NW = NC * NS
L = 16
NACC = 5

B, H, W, C = 8, 512, 512, 3
N_PIX = B * H * W
PIX_W = N_PIX // NW
HROWS_W = PIX_W // W
N_CHUNK = 8
CH_H = HROWS_W // N_CHUNK
CHUNK_PIX = CH_H * W
GROUPS = CHUNK_PIX // L
IROWS = CHUNK_PIX // 128

_params = pltpu.CompilerParams(use_tc_tiling_on_sc=True,
                               needs_layout_passes=False)


def _make_kernels():
    mesh = plsc.VectorSubcoreMesh(core_axis_name="c", subcore_axis_name="s",
                                  num_cores=NC, num_subcores=NS)

    @functools.partial(
        pl.kernel,
        out_type=jax.ShapeDtypeStruct((NW, 8, 128), jnp.float32),
        mesh=mesh,
        compiler_params=_params,
        scratch_types=[
            pltpu.VMEM((3 * CH_H, W), jnp.float32),
            pltpu.VMEM((3 * CH_H, W), jnp.float32),
            pltpu.VMEM((IROWS, 128), jnp.int32),
            pltpu.VMEM((IROWS, 128), jnp.int32),
            pltpu.VMEM((8, 128), jnp.float32),
            pltpu.SemaphoreType.DMA,
            pltpu.SemaphoreType.DMA,
            pltpu.SemaphoreType.DMA,
            pltpu.SemaphoreType.DMA,
        ],
    )
    def phase1(feats_hbm, inst_hbm, part_hbm, fbuf0, fbuf1, ibuf0, ibuf1,
               obuf, semf0, semf1, semi0, semi1):
        wid = lax.axis_index("s") * NC + lax.axis_index("c")
        b = wid // 4
        q = wid % 4
        h0 = q * HROWS_W
        fbase = b * (C * H) + h0
        irow0 = wid * (PIX_W // 128)
        fbuf = [fbuf0, fbuf1]
        ibuf = [ibuf0, ibuf1]
        semf = [semf0, semf1]
        semi = [semi0, semi1]

        def start(g, slot):
            hs = []
            for c in range(C):
                r = fbase + c * H + g * CH_H
                hs.append(pltpu.async_copy(
                    feats_hbm.at[pl.ds(r, CH_H), :],
                    fbuf[slot].at[pl.ds(c * CH_H, CH_H), :], semf[slot]))
            hs.append(pltpu.async_copy(
                inst_hbm.at[pl.ds(irow0 + g * IROWS, IROWS), :],
                ibuf[slot], semi[slot]))
            return hs

        pend = [None, None]
        pend[0] = start(0, 0)
        zeros = jnp.zeros((L,), jnp.float32)
        carry = (zeros, zeros, zeros, zeros, zeros)
        for g in range(N_CHUNK):
            slot = g % 2
            if g + 1 < N_CHUNK:
                pend[(g + 1) % 2] = start(g + 1, (g + 1) % 2)
            for hdl in pend[slot]:
                hdl.wait()
            fslot = fbuf[slot]
            islot = ibuf[slot]

            def body(i, acc, fslot=fslot, islot=islot):
                t0, t1, t2, c1, c2 = acc
                hr = i >> 5
                wc = pl.multiple_of((i & 31) << 4, 16)
                ir = i >> 3
                ic = pl.multiple_of((i & 7) << 4, 16)
                cv = islot[ir, pl.ds(ic, L)].astype(jnp.float32)
                t = (fslot[hr, pl.ds(wc, L)]
                     + fslot[hr + CH_H, pl.ds(wc, L)]
                     + fslot[hr + 2 * CH_H, pl.ds(wc, L)])
                x = t * cv
                return (t0 + t, t1 + x, t2 + x * cv, c1 + cv, c2 + cv * cv)

            carry = lax.fori_loop(0, GROUPS, body, carry, unroll=4)

        for a in range(NACC):
            obuf[0, pl.ds(a * L, L)] = carry[a]
        pltpu.sync_copy(obuf, part_hbm.at[wid])

    def phase2_body(iref, pref, oref):
        part = pref[...]
        s = jnp.sum(part[:, 0, :], axis=0)
        t0 = jnp.sum(s[0 * L:1 * L])
        t1 = jnp.sum(s[1 * L:2 * L])
        t2 = jnp.sum(s[2 * L:3 * L])
        c1 = jnp.sum(s[3 * L:4 * L])
        c2 = jnp.sum(s[4 * L:5 * L])
        n_elems = jnp.float32(N_PIX * 3)
        s2 = (t2 - t1) * 0.5
        s1 = t1 - 2.0 * s2
        s0 = t0 - s1 - s2
        n2 = 3.0 * (c2 - c1) * 0.5
        n1 = 3.0 * c1 - 2.0 * n2
        n0 = n_elems - n1 - n2
        m0 = s0 / n0
        m1 = s1 / n1
        m2 = s2 / n2
        cls = iref[...].reshape(H, W)
        v = jnp.where(cls == 0, m0,
                      jnp.where(cls == 1, m1, m2)).astype(jnp.float32)
        for c in range(C):
            oref[pl.ds(c * H, H), :] = v

    phase2 = pl.pallas_call(
        phase2_body,
        grid=(B,),
        in_specs=[
            pl.BlockSpec((H * W // 128, 128), lambda i: (i, 0)),
            pl.BlockSpec((NW, 8, 128), lambda i: (0, 0, 0)),
        ],
        out_specs=pl.BlockSpec((C * H, W), lambda i: (i, 0)),
        out_shape=jax.ShapeDtypeStruct((B * C * H, W), jnp.float32),
        compiler_params=pltpu.CompilerParams(
            dimension_semantics=("parallel",)),
    )

    def run(feats, inst):
        ft = feats.transpose(0, 3, 1, 2).reshape(B * C * H, W)
        ii = inst.reshape(N_PIX // 128, 128)
        part = phase1(ft, ii)
        out2d = phase2(ii, part)
        return out2d.reshape(B, C, H, W).transpose(0, 2, 3, 1)

    return run


_make_kernels_cached = functools.lru_cache(maxsize=None)(_make_kernels)


@jax.jit
def kernel(feats, inst):
    return _make_kernels_cached()(feats, inst)


# speedup vs baseline: 1.4715x; 1.0614x over previous
import functools

import jax
import jax.numpy as jnp
from jax import lax
from jax.experimental import pallas as pl
from jax.experimental.pallas import tpu as pltpu
from jax.experimental.pallas import tpu_sc as plsc

NC = 2
NS = 16
NW = NC * NS
L = 16
NACC = 5

B, H, W, C = 8, 512, 512, 3
N_PIX = B * H * W
SC_B = 6
TC_B = B - SC_B
PIX_W = SC_B * H * W // NW
HROWS_W = PIX_W // W
CH_H = 16
N_CHUNK = HROWS_W // CH_H
CHUNK_PIX = CH_H * W
GROUPS = CHUNK_PIX // L
IROWS = CHUNK_PIX // 128

_params = pltpu.CompilerParams(use_tc_tiling_on_sc=True,
                               needs_layout_passes=False)


def _make_kernels():
    mesh = plsc.VectorSubcoreMesh(core_axis_name="c", subcore_axis_name="s",
                                  num_cores=NC, num_subcores=NS)

    @functools.partial(
        pl.kernel,
        out_type=jax.ShapeDtypeStruct((NW, 8, 128), jnp.float32),
        mesh=mesh,
        compiler_params=_params,
        scratch_types=[
            pltpu.VMEM((3 * CH_H, W), jnp.float32),
            pltpu.VMEM((3 * CH_H, W), jnp.float32),
            pltpu.VMEM((IROWS, 128), jnp.int32),
            pltpu.VMEM((IROWS, 128), jnp.int32),
            pltpu.VMEM((8, 128), jnp.float32),
            pltpu.SemaphoreType.DMA,
            pltpu.SemaphoreType.DMA,
            pltpu.SemaphoreType.DMA,
            pltpu.SemaphoreType.DMA,
        ],
    )
    def phase1(feats_hbm, inst_hbm, part_hbm, fbuf0, fbuf1, ibuf0, ibuf1,
               obuf, semf0, semf1, semi0, semi1):
        wid = lax.axis_index("s") * NC + lax.axis_index("c")
        hh0 = wid * HROWS_W
        irow0 = wid * (PIX_W // 128)
        fbuf = [fbuf0, fbuf1]
        ibuf = [ibuf0, ibuf1]
        semf = [semf0, semf1]
        semi = [semi0, semi1]

        def start(g, slot):
            hh = hh0 + g * CH_H
            b = hh >> 9
            h = hh & (H - 1)
            hs = []
            for c in range(C):
                r = pl.multiple_of(b * (C * H) + c * H + h, CH_H)
                hs.append(pltpu.async_copy(
                    feats_hbm.at[pl.ds(r, CH_H), :],
                    fbuf[slot].at[pl.ds(c * CH_H, CH_H), :], semf[slot]))
            hs.append(pltpu.async_copy(
                inst_hbm.at[pl.ds(pl.multiple_of(irow0 + g * IROWS, IROWS),
                                  IROWS), :],
                ibuf[slot], semi[slot]))
            return hs

        pend = [None, None]
        pend[0] = start(0, 0)
        zeros = jnp.zeros((L,), jnp.float32)
        carry = (zeros, zeros, zeros, zeros, zeros)
        for g in range(N_CHUNK):
            slot = g % 2
            if g + 1 < N_CHUNK:
                pend[(g + 1) % 2] = start(g + 1, (g + 1) % 2)
            for hdl in pend[slot]:
                hdl.wait()
            fslot = fbuf[slot]
            islot = ibuf[slot]

            def body(i, acc, fslot=fslot, islot=islot):
                t0, t1, t2, c1, c2 = acc
                hr = i >> 5
                wc = pl.multiple_of((i & 31) << 4, 16)
                ir = i >> 3
                ic = pl.multiple_of((i & 7) << 4, 16)
                cv = islot[ir, pl.ds(ic, L)].astype(jnp.float32)
                t = (fslot[hr, pl.ds(wc, L)]
                     + fslot[hr + CH_H, pl.ds(wc, L)]
                     + fslot[hr + 2 * CH_H, pl.ds(wc, L)])
                x = t * cv
                return (t0 + t, t1 + x, t2 + x * cv, c1 + cv, c2 + cv * cv)

            carry = lax.fori_loop(0, GROUPS, body, carry, unroll=4)

        for a in range(NACC):
            obuf[0, pl.ds(a * L, L)] = carry[a]
        pltpu.sync_copy(obuf, part_hbm.at[wid])

    def phase1_tc_body(fref, iref, oref):
        i = pl.program_id(0)
        cls = iref[...].reshape(H, W).astype(jnp.float32)
        t = (fref[pl.ds(0, H), :] + fref[pl.ds(H, H), :]
             + fref[pl.ds(2 * H, H), :])
        x = t * cls
        t0 = jnp.sum(t)
        t1 = jnp.sum(x)
        t2 = jnp.sum(x * cls)
        c1 = jnp.sum(cls)
        c2 = jnp.sum(cls * cls)

        @pl.when(i == 0)
        def _():
            oref[...] = jnp.zeros((8, 128), jnp.float32)

        row = lax.broadcasted_iota(jnp.int32, (8, 128), 0)
        lane = lax.broadcasted_iota(jnp.int32, (8, 128), 1)
        vals = jnp.zeros((8, 128), jnp.float32)
        for k, val in enumerate((t0, t1, t2, c1, c2)):
            vals = jnp.where((row == 0) & (lane == k), val, vals)
        oref[...] = oref[...] + vals

    phase1_tc = pl.pallas_call(
        phase1_tc_body,
        grid=(TC_B,),
        in_specs=[
            pl.BlockSpec((C * H, W), lambda i: (SC_B + i, 0)),
            pl.BlockSpec((H * W // 128, 128), lambda i: (SC_B + i, 0)),
        ],
        out_specs=pl.BlockSpec((8, 128), lambda i: (0, 0)),
        out_shape=jax.ShapeDtypeStruct((8, 128), jnp.float32),
        compiler_params=pltpu.CompilerParams(
            dimension_semantics=("arbitrary",)),
    )

    def phase2_body(iref, pref, tpref, oref):
        part = pref[...]
        s = jnp.sum(part[:, 0, :], axis=0)
        t0 = jnp.sum(s[0 * L:1 * L]) + tpref[0, 0]
        t1 = jnp.sum(s[1 * L:2 * L]) + tpref[0, 1]
        t2 = jnp.sum(s[2 * L:3 * L]) + tpref[0, 2]
        c1 = jnp.sum(s[3 * L:4 * L]) + tpref[0, 3]
        c2 = jnp.sum(s[4 * L:5 * L]) + tpref[0, 4]
        n_elems = jnp.float32(N_PIX * 3)
        s2 = (t2 - t1) * 0.5
        s1 = t1 - 2.0 * s2
        s0 = t0 - s1 - s2
        n2 = 3.0 * (c2 - c1) * 0.5
        n1 = 3.0 * c1 - 2.0 * n2
        n0 = n_elems - n1 - n2
        m0 = s0 / n0
        m1 = s1 / n1
        m2 = s2 / n2
        cls = iref[...].reshape(H, W)
        v = jnp.where(cls == 0, m0,
                      jnp.where(cls == 1, m1, m2)).astype(jnp.float32)
        for c in range(C):
            oref[pl.ds(c * H, H), :] = v

    phase2 = pl.pallas_call(
        phase2_body,
        grid=(B,),
        in_specs=[
            pl.BlockSpec((H * W // 128, 128), lambda i: (i, 0)),
            pl.BlockSpec((NW, 8, 128), lambda i: (0, 0, 0)),
            pl.BlockSpec((8, 128), lambda i: (0, 0)),
        ],
        out_specs=pl.BlockSpec((C * H, W), lambda i: (i, 0)),
        out_shape=jax.ShapeDtypeStruct((B * C * H, W), jnp.float32),
        compiler_params=pltpu.CompilerParams(
            dimension_semantics=("parallel",)),
    )

    def run(feats, inst):
        ft = feats.transpose(0, 3, 1, 2).reshape(B * C * H, W)
        ii = inst.reshape(N_PIX // 128, 128)
        part = phase1(ft, ii)
        part_tc = phase1_tc(ft, ii)
        out2d = phase2(ii, part, part_tc)
        return out2d.reshape(B, C, H, W).transpose(0, 2, 3, 1)

    return run


_make_kernels_cached = functools.lru_cache(maxsize=None)(_make_kernels)


@jax.jit
def kernel(feats, inst):
    return _make_kernels_cached()(feats, inst)


# speedup vs baseline: 1.5014x; 1.0204x over previous
import functools

import jax
import jax.numpy as jnp
from jax import lax
from jax.experimental import pallas as pl
from jax.experimental.pallas import tpu as pltpu
from jax.experimental.pallas import tpu_sc as plsc

NC = 2
NS = 16
NW = NC * NS
L = 16
NACC = 5

B, H, W, C = 8, 512, 512, 3
N_PIX = B * H * W
SC_B = 5
TC_B = B - SC_B
PIX_W = SC_B * H * W // NW
HROWS_W = PIX_W // W
CH_H = 16
N_CHUNK = HROWS_W // CH_H
CHUNK_PIX = CH_H * W
GROUPS = CHUNK_PIX // L
IROWS = CHUNK_PIX // 128

_params = pltpu.CompilerParams(use_tc_tiling_on_sc=True,
                               needs_layout_passes=False)


def _make_kernels():
    mesh = plsc.VectorSubcoreMesh(core_axis_name="c", subcore_axis_name="s",
                                  num_cores=NC, num_subcores=NS)

    @functools.partial(
        pl.kernel,
        out_type=jax.ShapeDtypeStruct((NW, 8, 128), jnp.float32),
        mesh=mesh,
        compiler_params=_params,
        scratch_types=[
            pltpu.VMEM((3 * CH_H, W), jnp.float32),
            pltpu.VMEM((3 * CH_H, W), jnp.float32),
            pltpu.VMEM((IROWS, 128), jnp.int32),
            pltpu.VMEM((IROWS, 128), jnp.int32),
            pltpu.VMEM((8, 128), jnp.float32),
            pltpu.SemaphoreType.DMA,
            pltpu.SemaphoreType.DMA,
            pltpu.SemaphoreType.DMA,
            pltpu.SemaphoreType.DMA,
        ],
    )
    def phase1(feats_hbm, inst_hbm, part_hbm, fbuf0, fbuf1, ibuf0, ibuf1,
               obuf, semf0, semf1, semi0, semi1):
        wid = lax.axis_index("s") * NC + lax.axis_index("c")
        hh0 = wid * HROWS_W
        irow0 = wid * (PIX_W // 128)
        fbuf = [fbuf0, fbuf1]
        ibuf = [ibuf0, ibuf1]
        semf = [semf0, semf1]
        semi = [semi0, semi1]

        def start(g, slot):
            hh = hh0 + g * CH_H
            b = hh >> 9
            h = hh & (H - 1)
            hs = []
            for c in range(C):
                r = pl.multiple_of(b * (C * H) + c * H + h, CH_H)
                hs.append(pltpu.async_copy(
                    feats_hbm.at[pl.ds(r, CH_H), :],
                    fbuf[slot].at[pl.ds(c * CH_H, CH_H), :], semf[slot]))
            hs.append(pltpu.async_copy(
                inst_hbm.at[pl.ds(pl.multiple_of(irow0 + g * IROWS, IROWS),
                                  IROWS), :],
                ibuf[slot], semi[slot]))
            return hs

        pend = [None, None]
        pend[0] = start(0, 0)
        zeros = jnp.zeros((L,), jnp.float32)
        carry = (zeros, zeros, zeros, zeros, zeros)
        for g in range(N_CHUNK):
            slot = g % 2
            if g + 1 < N_CHUNK:
                pend[(g + 1) % 2] = start(g + 1, (g + 1) % 2)
            for hdl in pend[slot]:
                hdl.wait()
            fslot = fbuf[slot]
            islot = ibuf[slot]

            def body(i, acc, fslot=fslot, islot=islot):
                t0, t1, t2, c1, c2 = acc
                hr = i >> 5
                wc = pl.multiple_of((i & 31) << 4, 16)
                ir = i >> 3
                ic = pl.multiple_of((i & 7) << 4, 16)
                cv = islot[ir, pl.ds(ic, L)].astype(jnp.float32)
                t = (fslot[hr, pl.ds(wc, L)]
                     + fslot[hr + CH_H, pl.ds(wc, L)]
                     + fslot[hr + 2 * CH_H, pl.ds(wc, L)])
                x = t * cv
                return (t0 + t, t1 + x, t2 + x * cv, c1 + cv, c2 + cv * cv)

            carry = lax.fori_loop(0, GROUPS, body, carry, unroll=4)

        for a in range(NACC):
            obuf[0, pl.ds(a * L, L)] = carry[a]
        pltpu.sync_copy(obuf, part_hbm.at[wid])

    def phase1_tc_body(fref, iref, oref):
        i = pl.program_id(0)
        cls = iref[...].reshape(H, W).astype(jnp.float32)
        t = (fref[pl.ds(0, H), :] + fref[pl.ds(H, H), :]
             + fref[pl.ds(2 * H, H), :])
        x = t * cls
        t0 = jnp.sum(t)
        t1 = jnp.sum(x)
        t2 = jnp.sum(x * cls)
        c1 = jnp.sum(cls)
        c2 = jnp.sum(cls * cls)

        @pl.when(i == 0)
        def _():
            oref[...] = jnp.zeros((8, 128), jnp.float32)

        row = lax.broadcasted_iota(jnp.int32, (8, 128), 0)
        lane = lax.broadcasted_iota(jnp.int32, (8, 128), 1)
        vals = jnp.zeros((8, 128), jnp.float32)
        for k, val in enumerate((t0, t1, t2, c1, c2)):
            vals = jnp.where((row == 0) & (lane == k), val, vals)
        oref[...] = oref[...] + vals

    phase1_tc = pl.pallas_call(
        phase1_tc_body,
        grid=(TC_B,),
        in_specs=[
            pl.BlockSpec((C * H, W), lambda i: (SC_B + i, 0)),
            pl.BlockSpec((H * W // 128, 128), lambda i: (SC_B + i, 0)),
        ],
        out_specs=pl.BlockSpec((8, 128), lambda i: (0, 0)),
        out_shape=jax.ShapeDtypeStruct((8, 128), jnp.float32),
        compiler_params=pltpu.CompilerParams(
            dimension_semantics=("arbitrary",)),
    )

    def phase2_body(iref, pref, tpref, oref):
        part = pref[...]
        s = jnp.sum(part[:, 0, :], axis=0)
        t0 = jnp.sum(s[0 * L:1 * L]) + tpref[0, 0]
        t1 = jnp.sum(s[1 * L:2 * L]) + tpref[0, 1]
        t2 = jnp.sum(s[2 * L:3 * L]) + tpref[0, 2]
        c1 = jnp.sum(s[3 * L:4 * L]) + tpref[0, 3]
        c2 = jnp.sum(s[4 * L:5 * L]) + tpref[0, 4]
        n_elems = jnp.float32(N_PIX * 3)
        s2 = (t2 - t1) * 0.5
        s1 = t1 - 2.0 * s2
        s0 = t0 - s1 - s2
        n2 = 3.0 * (c2 - c1) * 0.5
        n1 = 3.0 * c1 - 2.0 * n2
        n0 = n_elems - n1 - n2
        m0 = s0 / n0
        m1 = s1 / n1
        m2 = s2 / n2
        cls = iref[...].reshape(H, W)
        v = jnp.where(cls == 0, m0,
                      jnp.where(cls == 1, m1, m2)).astype(jnp.float32)
        for c in range(C):
            oref[pl.ds(c * H, H), :] = v

    phase2 = pl.pallas_call(
        phase2_body,
        grid=(B,),
        in_specs=[
            pl.BlockSpec((H * W // 128, 128), lambda i: (i, 0)),
            pl.BlockSpec((NW, 8, 128), lambda i: (0, 0, 0)),
            pl.BlockSpec((8, 128), lambda i: (0, 0)),
        ],
        out_specs=pl.BlockSpec((C * H, W), lambda i: (i, 0)),
        out_shape=jax.ShapeDtypeStruct((B * C * H, W), jnp.float32),
        compiler_params=pltpu.CompilerParams(
            dimension_semantics=("parallel",)),
    )

    def run(feats, inst):
        ft = feats.transpose(0, 3, 1, 2).reshape(B * C * H, W)
        ii = inst.reshape(N_PIX // 128, 128)
        part = phase1(ft, ii)
        part_tc = phase1_tc(ft, ii)
        out2d = phase2(ii, part, part_tc)
        return out2d.reshape(B, C, H, W).transpose(0, 2, 3, 1)

    return run


_make_kernels_cached = functools.lru_cache(maxsize=None)(_make_kernels)


@jax.jit
def kernel(feats, inst):
    return _make_kernels_cached()(feats, inst)


# speedup vs baseline: 1.5435x; 1.0280x over previous
import functools

import jax
import jax.numpy as jnp
from jax import lax
from jax.experimental import pallas as pl
from jax.experimental.pallas import tpu as pltpu
from jax.experimental.pallas import tpu_sc as plsc

NC = 2
NS = 16
NW = NC * NS
L = 16
NACC = 5

B, H, W, C = 8, 512, 512, 3
N_PIX = B * H * W
SC_B = 4
TC_B = B - SC_B
PIX_W = SC_B * H * W // NW
HROWS_W = PIX_W // W
CH_H = 16
N_CHUNK = HROWS_W // CH_H
CHUNK_PIX = CH_H * W
GROUPS = CHUNK_PIX // L
IROWS = CHUNK_PIX // 128

_params = pltpu.CompilerParams(use_tc_tiling_on_sc=True,
                               needs_layout_passes=False)


def _make_kernels():
    mesh = plsc.VectorSubcoreMesh(core_axis_name="c", subcore_axis_name="s",
                                  num_cores=NC, num_subcores=NS)

    @functools.partial(
        pl.kernel,
        out_type=jax.ShapeDtypeStruct((NW, 8, 128), jnp.float32),
        mesh=mesh,
        compiler_params=_params,
        scratch_types=[
            pltpu.VMEM((3 * CH_H, W), jnp.float32),
            pltpu.VMEM((3 * CH_H, W), jnp.float32),
            pltpu.VMEM((IROWS, 128), jnp.int32),
            pltpu.VMEM((IROWS, 128), jnp.int32),
            pltpu.VMEM((8, 128), jnp.float32),
            pltpu.SemaphoreType.DMA,
            pltpu.SemaphoreType.DMA,
            pltpu.SemaphoreType.DMA,
            pltpu.SemaphoreType.DMA,
        ],
    )
    def phase1(feats_hbm, inst_hbm, part_hbm, fbuf0, fbuf1, ibuf0, ibuf1,
               obuf, semf0, semf1, semi0, semi1):
        wid = lax.axis_index("s") * NC + lax.axis_index("c")
        hh0 = wid * HROWS_W
        irow0 = wid * (PIX_W // 128)
        fbuf = [fbuf0, fbuf1]
        ibuf = [ibuf0, ibuf1]
        semf = [semf0, semf1]
        semi = [semi0, semi1]

        def start(g, slot):
            hh = hh0 + g * CH_H
            b = hh >> 9
            h = hh & (H - 1)
            hs = []
            for c in range(C):
                r = pl.multiple_of(b * (C * H) + c * H + h, CH_H)
                hs.append(pltpu.async_copy(
                    feats_hbm.at[pl.ds(r, CH_H), :],
                    fbuf[slot].at[pl.ds(c * CH_H, CH_H), :], semf[slot]))
            hs.append(pltpu.async_copy(
                inst_hbm.at[pl.ds(pl.multiple_of(irow0 + g * IROWS, IROWS),
                                  IROWS), :],
                ibuf[slot], semi[slot]))
            return hs

        pend = [None, None]
        pend[0] = start(0, 0)
        zeros = jnp.zeros((L,), jnp.float32)
        carry = (zeros, zeros, zeros, zeros, zeros)
        for g in range(N_CHUNK):
            slot = g % 2
            if g + 1 < N_CHUNK:
                pend[(g + 1) % 2] = start(g + 1, (g + 1) % 2)
            for hdl in pend[slot]:
                hdl.wait()
            fslot = fbuf[slot]
            islot = ibuf[slot]

            def body(i, acc, fslot=fslot, islot=islot):
                t0, t1, t2, c1, c2 = acc
                hr = i >> 5
                wc = pl.multiple_of((i & 31) << 4, 16)
                ir = i >> 3
                ic = pl.multiple_of((i & 7) << 4, 16)
                cv = islot[ir, pl.ds(ic, L)].astype(jnp.float32)
                t = (fslot[hr, pl.ds(wc, L)]
                     + fslot[hr + CH_H, pl.ds(wc, L)]
                     + fslot[hr + 2 * CH_H, pl.ds(wc, L)])
                x = t * cv
                return (t0 + t, t1 + x, t2 + x * cv, c1 + cv, c2 + cv * cv)

            carry = lax.fori_loop(0, GROUPS, body, carry, unroll=4)

        for a in range(NACC):
            obuf[0, pl.ds(a * L, L)] = carry[a]
        pltpu.sync_copy(obuf, part_hbm.at[wid])

    def phase1_tc_body(fref, iref, oref):
        i = pl.program_id(0)
        cls = iref[...].reshape(H, W).astype(jnp.float32)
        t = (fref[pl.ds(0, H), :] + fref[pl.ds(H, H), :]
             + fref[pl.ds(2 * H, H), :])
        x = t * cls
        t0 = jnp.sum(t)
        t1 = jnp.sum(x)
        t2 = jnp.sum(x * cls)
        c1 = jnp.sum(cls)
        c2 = jnp.sum(cls * cls)

        @pl.when(i == 0)
        def _():
            oref[...] = jnp.zeros((8, 128), jnp.float32)

        row = lax.broadcasted_iota(jnp.int32, (8, 128), 0)
        lane = lax.broadcasted_iota(jnp.int32, (8, 128), 1)
        vals = jnp.zeros((8, 128), jnp.float32)
        for k, val in enumerate((t0, t1, t2, c1, c2)):
            vals = jnp.where((row == 0) & (lane == k), val, vals)
        oref[...] = oref[...] + vals

    phase1_tc = pl.pallas_call(
        phase1_tc_body,
        grid=(TC_B,),
        in_specs=[
            pl.BlockSpec((C * H, W), lambda i: (SC_B + i, 0)),
            pl.BlockSpec((H * W // 128, 128), lambda i: (SC_B + i, 0)),
        ],
        out_specs=pl.BlockSpec((8, 128), lambda i: (0, 0)),
        out_shape=jax.ShapeDtypeStruct((8, 128), jnp.float32),
        compiler_params=pltpu.CompilerParams(
            dimension_semantics=("arbitrary",)),
    )

    def phase2_body(iref, pref, tpref, oref):
        part = pref[...]
        s = jnp.sum(part[:, 0, :], axis=0)
        t0 = jnp.sum(s[0 * L:1 * L]) + tpref[0, 0]
        t1 = jnp.sum(s[1 * L:2 * L]) + tpref[0, 1]
        t2 = jnp.sum(s[2 * L:3 * L]) + tpref[0, 2]
        c1 = jnp.sum(s[3 * L:4 * L]) + tpref[0, 3]
        c2 = jnp.sum(s[4 * L:5 * L]) + tpref[0, 4]
        n_elems = jnp.float32(N_PIX * 3)
        s2 = (t2 - t1) * 0.5
        s1 = t1 - 2.0 * s2
        s0 = t0 - s1 - s2
        n2 = 3.0 * (c2 - c1) * 0.5
        n1 = 3.0 * c1 - 2.0 * n2
        n0 = n_elems - n1 - n2
        m0 = s0 / n0
        m1 = s1 / n1
        m2 = s2 / n2
        cls = iref[...].reshape(H, W)
        v = jnp.where(cls == 0, m0,
                      jnp.where(cls == 1, m1, m2)).astype(jnp.float32)
        for c in range(C):
            oref[pl.ds(c * H, H), :] = v

    phase2 = pl.pallas_call(
        phase2_body,
        grid=(B,),
        in_specs=[
            pl.BlockSpec((H * W // 128, 128), lambda i: (i, 0)),
            pl.BlockSpec((NW, 8, 128), lambda i: (0, 0, 0)),
            pl.BlockSpec((8, 128), lambda i: (0, 0)),
        ],
        out_specs=pl.BlockSpec((C * H, W), lambda i: (i, 0)),
        out_shape=jax.ShapeDtypeStruct((B * C * H, W), jnp.float32),
        compiler_params=pltpu.CompilerParams(
            dimension_semantics=("parallel",)),
    )

    def run(feats, inst):
        ft = feats.transpose(0, 3, 1, 2).reshape(B * C * H, W)
        ii = inst.reshape(N_PIX // 128, 128)
        part = phase1(ft, ii)
        part_tc = phase1_tc(ft, ii)
        out2d = phase2(ii, part, part_tc)
        return out2d.reshape(B, C, H, W).transpose(0, 2, 3, 1)

    return run


_make_kernels_cached = functools.lru_cache(maxsize=None)(_make_kernels)


@jax.jit
def kernel(feats, inst):
    return _make_kernels_cached()(feats, inst)


# speedup vs baseline: 1.5468x; 1.0021x over previous
import functools

import jax
import jax.numpy as jnp
from jax import lax
from jax.experimental import pallas as pl
from jax.experimental.pallas import tpu as pltpu
from jax.experimental.pallas import tpu_sc as plsc

NC = 2
NS = 16
NW = NC * NS
L = 16
NACC = 5

B, H, W, C = 8, 512, 512, 3
N_PIX = B * H * W
SC_B = 3
TC_B = B - SC_B
PIX_W = SC_B * H * W // NW
HROWS_W = PIX_W // W
CH_H = 16
N_CHUNK = HROWS_W // CH_H
CHUNK_PIX = CH_H * W
GROUPS = CHUNK_PIX // L
IROWS = CHUNK_PIX // 128

_params = pltpu.CompilerParams(use_tc_tiling_on_sc=True,
                               needs_layout_passes=False)


def _make_kernels():
    mesh = plsc.VectorSubcoreMesh(core_axis_name="c", subcore_axis_name="s",
                                  num_cores=NC, num_subcores=NS)

    @functools.partial(
        pl.kernel,
        out_type=jax.ShapeDtypeStruct((NW, 8, 128), jnp.float32),
        mesh=mesh,
        compiler_params=_params,
        scratch_types=[
            pltpu.VMEM((3 * CH_H, W), jnp.float32),
            pltpu.VMEM((3 * CH_H, W), jnp.float32),
            pltpu.VMEM((IROWS, 128), jnp.int32),
            pltpu.VMEM((IROWS, 128), jnp.int32),
            pltpu.VMEM((8, 128), jnp.float32),
            pltpu.SemaphoreType.DMA,
            pltpu.SemaphoreType.DMA,
            pltpu.SemaphoreType.DMA,
            pltpu.SemaphoreType.DMA,
        ],
    )
    def phase1(feats_hbm, inst_hbm, part_hbm, fbuf0, fbuf1, ibuf0, ibuf1,
               obuf, semf0, semf1, semi0, semi1):
        wid = lax.axis_index("s") * NC + lax.axis_index("c")
        hh0 = wid * HROWS_W
        irow0 = wid * (PIX_W // 128)
        fbuf = [fbuf0, fbuf1]
        ibuf = [ibuf0, ibuf1]
        semf = [semf0, semf1]
        semi = [semi0, semi1]

        def start(g, slot):
            hh = hh0 + g * CH_H
            b = hh >> 9
            h = hh & (H - 1)
            hs = []
            for c in range(C):
                r = pl.multiple_of(b * (C * H) + c * H + h, CH_H)
                hs.append(pltpu.async_copy(
                    feats_hbm.at[pl.ds(r, CH_H), :],
                    fbuf[slot].at[pl.ds(c * CH_H, CH_H), :], semf[slot]))
            hs.append(pltpu.async_copy(
                inst_hbm.at[pl.ds(pl.multiple_of(irow0 + g * IROWS, IROWS),
                                  IROWS), :],
                ibuf[slot], semi[slot]))
            return hs

        pend = [None, None]
        pend[0] = start(0, 0)
        zeros = jnp.zeros((L,), jnp.float32)
        carry = (zeros, zeros, zeros, zeros, zeros)
        for g in range(N_CHUNK):
            slot = g % 2
            if g + 1 < N_CHUNK:
                pend[(g + 1) % 2] = start(g + 1, (g + 1) % 2)
            for hdl in pend[slot]:
                hdl.wait()
            fslot = fbuf[slot]
            islot = ibuf[slot]

            def body(i, acc, fslot=fslot, islot=islot):
                t0, t1, t2, c1, c2 = acc
                hr = i >> 5
                wc = pl.multiple_of((i & 31) << 4, 16)
                ir = i >> 3
                ic = pl.multiple_of((i & 7) << 4, 16)
                cv = islot[ir, pl.ds(ic, L)].astype(jnp.float32)
                t = (fslot[hr, pl.ds(wc, L)]
                     + fslot[hr + CH_H, pl.ds(wc, L)]
                     + fslot[hr + 2 * CH_H, pl.ds(wc, L)])
                x = t * cv
                return (t0 + t, t1 + x, t2 + x * cv, c1 + cv, c2 + cv * cv)

            carry = lax.fori_loop(0, GROUPS, body, carry, unroll=4)

        for a in range(NACC):
            obuf[0, pl.ds(a * L, L)] = carry[a]
        pltpu.sync_copy(obuf, part_hbm.at[wid])

    def phase1_tc_body(fref, iref, oref):
        i = pl.program_id(0)
        cls = iref[...].reshape(H, W).astype(jnp.float32)
        t = (fref[pl.ds(0, H), :] + fref[pl.ds(H, H), :]
             + fref[pl.ds(2 * H, H), :])
        x = t * cls
        t0 = jnp.sum(t)
        t1 = jnp.sum(x)
        t2 = jnp.sum(x * cls)
        c1 = jnp.sum(cls)
        c2 = jnp.sum(cls * cls)

        @pl.when(i == 0)
        def _():
            oref[...] = jnp.zeros((8, 128), jnp.float32)

        row = lax.broadcasted_iota(jnp.int32, (8, 128), 0)
        lane = lax.broadcasted_iota(jnp.int32, (8, 128), 1)
        vals = jnp.zeros((8, 128), jnp.float32)
        for k, val in enumerate((t0, t1, t2, c1, c2)):
            vals = jnp.where((row == 0) & (lane == k), val, vals)
        oref[...] = oref[...] + vals

    phase1_tc = pl.pallas_call(
        phase1_tc_body,
        grid=(TC_B,),
        in_specs=[
            pl.BlockSpec((C * H, W), lambda i: (SC_B + i, 0)),
            pl.BlockSpec((H * W // 128, 128), lambda i: (SC_B + i, 0)),
        ],
        out_specs=pl.BlockSpec((8, 128), lambda i: (0, 0)),
        out_shape=jax.ShapeDtypeStruct((8, 128), jnp.float32),
        compiler_params=pltpu.CompilerParams(
            dimension_semantics=("arbitrary",)),
    )

    def phase2_body(iref, pref, tpref, oref):
        part = pref[...]
        s = jnp.sum(part[:, 0, :], axis=0)
        t0 = jnp.sum(s[0 * L:1 * L]) + tpref[0, 0]
        t1 = jnp.sum(s[1 * L:2 * L]) + tpref[0, 1]
        t2 = jnp.sum(s[2 * L:3 * L]) + tpref[0, 2]
        c1 = jnp.sum(s[3 * L:4 * L]) + tpref[0, 3]
        c2 = jnp.sum(s[4 * L:5 * L]) + tpref[0, 4]
        n_elems = jnp.float32(N_PIX * 3)
        s2 = (t2 - t1) * 0.5
        s1 = t1 - 2.0 * s2
        s0 = t0 - s1 - s2
        n2 = 3.0 * (c2 - c1) * 0.5
        n1 = 3.0 * c1 - 2.0 * n2
        n0 = n_elems - n1 - n2
        m0 = s0 / n0
        m1 = s1 / n1
        m2 = s2 / n2
        cls = iref[...].reshape(H, W)
        v = jnp.where(cls == 0, m0,
                      jnp.where(cls == 1, m1, m2)).astype(jnp.float32)
        for c in range(C):
            oref[pl.ds(c * H, H), :] = v

    phase2 = pl.pallas_call(
        phase2_body,
        grid=(B,),
        in_specs=[
            pl.BlockSpec((H * W // 128, 128), lambda i: (i, 0)),
            pl.BlockSpec((NW, 8, 128), lambda i: (0, 0, 0)),
            pl.BlockSpec((8, 128), lambda i: (0, 0)),
        ],
        out_specs=pl.BlockSpec((C * H, W), lambda i: (i, 0)),
        out_shape=jax.ShapeDtypeStruct((B * C * H, W), jnp.float32),
        compiler_params=pltpu.CompilerParams(
            dimension_semantics=("parallel",)),
    )

    def run(feats, inst):
        ft = feats.transpose(0, 3, 1, 2).reshape(B * C * H, W)
        ii = inst.reshape(N_PIX // 128, 128)
        part = phase1(ft, ii)
        part_tc = phase1_tc(ft, ii)
        out2d = phase2(ii, part, part_tc)
        return out2d.reshape(B, C, H, W).transpose(0, 2, 3, 1)

    return run


_make_kernels_cached = functools.lru_cache(maxsize=None)(_make_kernels)


@jax.jit
def kernel(feats, inst):
    return _make_kernels_cached()(feats, inst)
